# baseline (speedup 1.0000x reference)
import jax
import jax.numpy as jnp
from jax.experimental import pallas as pl
from jax.experimental.pallas import tpu as pltpu

_N = 4096
_NFEAT = 128
_NHID = 64
_HID2 = 32
_BLK = 1024
_G = _N // _BLK


def _layer1_kernel(x_ref, adj_ref, w1_ref, b1_ref, w2_ref, s2_ref, s1_scr, h_scr):
    i = pl.program_id(0)

    @pl.when(i == 0)
    def _():
        s1 = jnp.dot(x_ref[...], w1_ref[...],
                     preferred_element_type=jnp.float32)
        s1_scr[...] = s1.astype(jnp.bfloat16)

    adj_bf = adj_ref[...].astype(jnp.bfloat16)
    h = jnp.dot(adj_bf, s1_scr[...],
                preferred_element_type=jnp.float32) + b1_ref[...]
    h_scr[pl.ds(i * _BLK, _BLK), :] = jnp.maximum(h, 0.0).astype(jnp.bfloat16)

    @pl.when(i == _G - 1)
    def _():
        s2_ref[...] = jnp.dot(h_scr[...], w2_ref[...].astype(jnp.bfloat16),
                              preferred_element_type=jnp.float32)


def _layer2_kernel(adj_ref, s2_ref, b2_ref, z_ref):
    adj_bf = adj_ref[...].astype(jnp.bfloat16)
    z_ref[...] = jnp.dot(adj_bf, s2_ref[...],
                         preferred_element_type=jnp.float32) + b2_ref[...]


def _decoder_kernel(z_ref, zt_ref, recon_ref):
    recon_ref[...] = jnp.dot(z_ref[...], zt_ref[...],
                             preferred_element_type=jnp.float32)


def kernel(x, adj, W1, b1, W2, b2):
    b1r = b1.reshape(1, _NHID)
    b2r = b2.reshape(1, _HID2)

    s2 = pl.pallas_call(
        _layer1_kernel,
        grid=(_G,),
        in_specs=[
            pl.BlockSpec((_N, _NFEAT), lambda i: (0, 0)),
            pl.BlockSpec((_BLK, _N), lambda i: (i, 0)),
            pl.BlockSpec((_NFEAT, _NHID), lambda i: (0, 0)),
            pl.BlockSpec((1, _NHID), lambda i: (0, 0)),
            pl.BlockSpec((_NHID, _HID2), lambda i: (0, 0)),
        ],
        out_specs=pl.BlockSpec((_N, _HID2), lambda i: (0, 0)),
        out_shape=jax.ShapeDtypeStruct((_N, _HID2), jnp.float32),
        scratch_shapes=[
            pltpu.VMEM((_N, _NHID), jnp.bfloat16),
            pltpu.VMEM((_N, _NHID), jnp.bfloat16),
        ],
        compiler_params=pltpu.CompilerParams(
            dimension_semantics=("arbitrary",)),
    )(x, adj, W1, b1r, W2)

    z = pl.pallas_call(
        _layer2_kernel,
        grid=(_G,),
        in_specs=[
            pl.BlockSpec((_BLK, _N), lambda i: (i, 0)),
            pl.BlockSpec((_N, _HID2), lambda i: (0, 0)),
            pl.BlockSpec((1, _HID2), lambda i: (0, 0)),
        ],
        out_specs=pl.BlockSpec((_BLK, _HID2), lambda i: (i, 0)),
        out_shape=jax.ShapeDtypeStruct((_N, _HID2), jnp.float32),
        compiler_params=pltpu.CompilerParams(
            dimension_semantics=("arbitrary",)),
    )(adj, s2.astype(jnp.bfloat16), b2r)

    z_bf = z.astype(jnp.bfloat16)
    zt = z_bf.T
    recon = pl.pallas_call(
        _decoder_kernel,
        grid=(_G,),
        in_specs=[
            pl.BlockSpec((_BLK, _HID2), lambda i: (i, 0)),
            pl.BlockSpec((_HID2, _N), lambda i: (0, 0)),
        ],
        out_specs=pl.BlockSpec((_BLK, _N), lambda i: (i, 0)),
        out_shape=jax.ShapeDtypeStruct((_N, _N), jnp.float32),
        compiler_params=pltpu.CompilerParams(
            dimension_semantics=("arbitrary",)),
    )(z_bf, zt)

    return (recon, z)


# speedup vs baseline: 1.0540x; 1.0540x over previous
import jax
import jax.numpy as jnp
from jax.experimental import pallas as pl
from jax.experimental.pallas import tpu as pltpu

_N = 4096
_NFEAT = 128
_NHID = 64
_HID2 = 32
_BLK = 512
_G = _N // _BLK


def _layer1_kernel(x_ref, adj_ref, w1_ref, b1_ref, w2_ref, s2_ref, s1_scr, h_scr):
    i = pl.program_id(0)

    @pl.when(i == 0)
    def _():
        s1 = jnp.dot(x_ref[...], w1_ref[...],
                     preferred_element_type=jnp.float32)
        s1_scr[...] = s1.astype(jnp.bfloat16)

    adj_bf = adj_ref[...].astype(jnp.bfloat16)
    h = jnp.dot(adj_bf, s1_scr[...],
                preferred_element_type=jnp.float32) + b1_ref[...]
    h_scr[pl.ds(i * _BLK, _BLK), :] = jnp.maximum(h, 0.0).astype(jnp.bfloat16)

    @pl.when(i == _G - 1)
    def _():
        s2_ref[...] = jnp.dot(h_scr[...], w2_ref[...].astype(jnp.bfloat16),
                              preferred_element_type=jnp.float32)


def _layer2_kernel(adj_ref, s2_ref, b2_ref, z_ref):
    adj_bf = adj_ref[...].astype(jnp.bfloat16)
    z_ref[...] = jnp.dot(adj_bf, s2_ref[...],
                         preferred_element_type=jnp.float32) + b2_ref[...]


def _decoder_kernel(z_ref, zt_ref, recon_ref):
    recon_ref[...] = jnp.dot(z_ref[...], zt_ref[...],
                             preferred_element_type=jnp.float32)


def kernel(x, adj, W1, b1, W2, b2):
    b1r = b1.reshape(1, _NHID)
    b2r = b2.reshape(1, _HID2)

    s2 = pl.pallas_call(
        _layer1_kernel,
        grid=(_G,),
        in_specs=[
            pl.BlockSpec((_N, _NFEAT), lambda i: (0, 0)),
            pl.BlockSpec((_BLK, _N), lambda i: (i, 0)),
            pl.BlockSpec((_NFEAT, _NHID), lambda i: (0, 0)),
            pl.BlockSpec((1, _NHID), lambda i: (0, 0)),
            pl.BlockSpec((_NHID, _HID2), lambda i: (0, 0)),
        ],
        out_specs=pl.BlockSpec((_N, _HID2), lambda i: (0, 0)),
        out_shape=jax.ShapeDtypeStruct((_N, _HID2), jnp.float32),
        scratch_shapes=[
            pltpu.VMEM((_N, _NHID), jnp.bfloat16),
            pltpu.VMEM((_N, _NHID), jnp.bfloat16),
        ],
        compiler_params=pltpu.CompilerParams(
            dimension_semantics=("arbitrary",)),
    )(x, adj, W1, b1r, W2)

    z = pl.pallas_call(
        _layer2_kernel,
        grid=(_G,),
        in_specs=[
            pl.BlockSpec((_BLK, _N), lambda i: (i, 0)),
            pl.BlockSpec((_N, _HID2), lambda i: (0, 0)),
            pl.BlockSpec((1, _HID2), lambda i: (0, 0)),
        ],
        out_specs=pl.BlockSpec((_BLK, _HID2), lambda i: (i, 0)),
        out_shape=jax.ShapeDtypeStruct((_N, _HID2), jnp.float32),
        compiler_params=pltpu.CompilerParams(
            dimension_semantics=("parallel",)),
    )(adj, s2.astype(jnp.bfloat16), b2r)

    z_bf = z.astype(jnp.bfloat16)
    zt = z_bf.T
    recon = pl.pallas_call(
        _decoder_kernel,
        grid=(_G,),
        in_specs=[
            pl.BlockSpec((_BLK, _HID2), lambda i: (i, 0)),
            pl.BlockSpec((_HID2, _N), lambda i: (0, 0)),
        ],
        out_specs=pl.BlockSpec((_BLK, _N), lambda i: (i, 0)),
        out_shape=jax.ShapeDtypeStruct((_N, _N), jnp.float32),
        compiler_params=pltpu.CompilerParams(
            dimension_semantics=("parallel",)),
    )(z_bf, zt)

    return (recon, z)


# speedup vs baseline: 1.2276x; 1.1646x over previous
import jax
import jax.numpy as jnp
from jax.experimental import pallas as pl
from jax.experimental.pallas import tpu as pltpu

_N = 4096
_NFEAT = 128
_NHID = 64
_HID2 = 32
_BLK = 256
_G = _N // _BLK


def _gcn_kernel(x_ref, adj_ref, w1_ref, b1_ref, w2_ref, b2_ref,
                z_ref, recon_ref,
                adj_scr, s1_scr, h_scr, s2_scr, zbf_scr, zt_scr):
    i = pl.program_id(0)

    @pl.when(i == 0)
    def _():
        s1 = jnp.dot(x_ref[...], w1_ref[...],
                     preferred_element_type=jnp.float32)
        s1_scr[...] = s1.astype(jnp.bfloat16)

    @pl.when(i < _G)
    def _():
        a = adj_ref[...].astype(jnp.bfloat16)
        adj_scr[pl.ds(i * _BLK, _BLK), :] = a
        h = jnp.dot(a, s1_scr[...],
                    preferred_element_type=jnp.float32) + b1_ref[...]
        h_scr[pl.ds(i * _BLK, _BLK), :] = jnp.maximum(h, 0.0).astype(jnp.bfloat16)

    @pl.when(i == _G - 1)
    def _():
        s2 = jnp.dot(h_scr[...], w2_ref[...].astype(jnp.bfloat16),
                     preferred_element_type=jnp.float32)
        s2_scr[...] = s2.astype(jnp.bfloat16)

    @pl.when(jnp.logical_and(i >= _G, i < 2 * _G))
    def _():
        j = i - _G
        zj = jnp.dot(adj_scr[pl.ds(j * _BLK, _BLK), :], s2_scr[...],
                     preferred_element_type=jnp.float32) + b2_ref[...]
        z_ref[...] = zj
        zj_bf = zj.astype(jnp.bfloat16)
        zbf_scr[pl.ds(j * _BLK, _BLK), :] = zj_bf
        zt_scr[:, pl.ds(j * _BLK, _BLK)] = zj_bf.T

    @pl.when(i >= 2 * _G)
    def _():
        k = i - 2 * _G
        recon_ref[...] = jnp.dot(zbf_scr[pl.ds(k * _BLK, _BLK), :],
                                 zt_scr[...],
                                 preferred_element_type=jnp.float32)


def kernel(x, adj, W1, b1, W2, b2):
    b1r = b1.reshape(1, _NHID)
    b2r = b2.reshape(1, _HID2)

    z, recon = pl.pallas_call(
        _gcn_kernel,
        grid=(3 * _G,),
        in_specs=[
            pl.BlockSpec((_N, _NFEAT), lambda i: (0, 0)),
            pl.BlockSpec((_BLK, _N), lambda i: (jnp.minimum(i, _G - 1), 0)),
            pl.BlockSpec((_NFEAT, _NHID), lambda i: (0, 0)),
            pl.BlockSpec((1, _NHID), lambda i: (0, 0)),
            pl.BlockSpec((_NHID, _HID2), lambda i: (0, 0)),
            pl.BlockSpec((1, _HID2), lambda i: (0, 0)),
        ],
        out_specs=[
            pl.BlockSpec((_BLK, _HID2),
                         lambda i: (jnp.clip(i - _G, 0, _G - 1), 0)),
            pl.BlockSpec((_BLK, _N),
                         lambda i: (jnp.clip(i - 2 * _G, 0, _G - 1), 0)),
        ],
        out_shape=[
            jax.ShapeDtypeStruct((_N, _HID2), jnp.float32),
            jax.ShapeDtypeStruct((_N, _N), jnp.float32),
        ],
        scratch_shapes=[
            pltpu.VMEM((_N, _N), jnp.bfloat16),
            pltpu.VMEM((_N, _NHID), jnp.bfloat16),
            pltpu.VMEM((_N, _NHID), jnp.bfloat16),
            pltpu.VMEM((_N, _HID2), jnp.bfloat16),
            pltpu.VMEM((_N, _HID2), jnp.bfloat16),
            pltpu.VMEM((_HID2, _N), jnp.bfloat16),
        ],
        compiler_params=pltpu.CompilerParams(
            dimension_semantics=("arbitrary",)),
    )(x, adj, W1, b1r, W2, b2r)

    return (recon, z)
